# baseline (speedup 1.0000x reference)
import functools

import jax
import jax.numpy as jnp
from jax import lax
from jax.experimental import pallas as pl
from jax.experimental.pallas import tpu as pltpu
from jax.experimental.pallas import tpu_sc as plsc

_B = 16384
_M = 14
_D = 64
_NC = 2
_NS = 16
_L = 16
_NW = _NC * _NS
_PER_W = _B // _NW
_C = 32
_NCHUNK = _PER_W // _C
_NG = _C // _L
_RPQ = _M - 1
_TS = 1024


def _row_of(rv):
    return ((rv >> 10) << 9) + (rv & 511)


def _tr_body(src, dst):
    x = src[...]
    xt = jnp.swapaxes(x, 0, 1)
    dst[...] = jnp.concatenate([xt[: _TS // 2], xt[_TS // 2:]], axis=1)


def _to_rows128(table_t):
    n = table_t.shape[1]
    grid = (n + _TS - 1) // _TS
    return pl.pallas_call(
        _tr_body,
        grid=(grid,),
        in_specs=[pl.BlockSpec((_D, _TS), lambda i: (0, i))],
        out_specs=pl.BlockSpec((_TS // 2, 128), lambda i: (i, 0)),
        out_shape=jax.ShapeDtypeStruct((grid * (_TS // 2), 128), jnp.float32),
    )(table_t)


def _impact_body(uids, iids, users, items, nbs, out,
                 uidx_all, iidx_all, nb_all, out_all, eidx,
                 u0, u1, e0, e1, sem_nb, sem0, sem1):
    wid = lax.axis_index("s") * _NC + lax.axis_index("c")
    base0 = wid * _PER_W
    iota = lax.iota(jnp.int32, _L)
    ubufs = (u0, u1)
    ebufs = (e0, e1)
    sems = (sem0, sem1)

    pltpu.sync_copy(uids.at[pl.ds(base0, _PER_W)], uidx_all)
    pltpu.sync_copy(iids.at[pl.ds(base0, _PER_W)], iidx_all)
    nbcp = pltpu.async_copy(nbs.at[iidx_all], nb_all, sem_nb)

    _SLOT = _RPQ * _C + _C

    def issue(n, s):
        for g in range(_NG):
            iv = iidx_all[pl.ds(n * _C + g * _L, _L)] * _M
            for t in range(_RPQ):
                eidx[pl.ds(s * _SLOT + t * _C + g * _L, _L)] = \
                    _row_of(iv + (t + 1))
            uv = uidx_all[pl.ds(n * _C + g * _L, _L)]
            eidx[pl.ds(s * _SLOT + _RPQ * _C + g * _L, _L)] = _row_of(uv)
        pltpu.async_copy(items.at[eidx.at[pl.ds(s * _SLOT, _RPQ * _C)]],
                         ebufs[s], sems[s])
        pltpu.async_copy(users.at[eidx.at[pl.ds(s * _SLOT + _RPQ * _C, _C)]],
                         ubufs[s], sems[s])

    def drain(s):
        pltpu.make_async_copy(items.at[eidx.at[pl.ds(0, _RPQ * _C)]],
                              ebufs[s], sems[s]).wait()
        pltpu.make_async_copy(users.at[eidx.at[pl.ds(0, _C)]],
                              ubufs[s], sems[s]).wait()

    def compute(n, s):
        urows_v = ubufs[s]
        erows_v = ebufs[s]
        for g in range(_NG):
            rows = iota + (g * _L)
            nb = nb_all[pl.ds(n * _C + g * _L, _L)]
            uv = uidx_all[pl.ds(n * _C + g * _L, _L)]
            uhalf = ((uv >> 9) & 1) << 6
            iv = iidx_all[pl.ds(n * _C + g * _L, _L)] * _M
            halfs = [((iv + j >> 9) & 1) << 6 for j in range(1, _M)]

            def cstep(c, accs, rows=rows, uhalf=uhalf, halfs=halfs,
                      urows_v=urows_v, erows_v=erows_v):
                csplat = jnp.full((_L,), 0, jnp.int32) + c
                u_c = plsc.load_gather(urows_v, [rows, uhalf + csplat])
                new = []
                for j in range(1, _M):
                    e = plsc.load_gather(
                        erows_v,
                        [rows + ((j - 1) * _C), halfs[j - 1] + csplat])
                    dv = u_c - e
                    new.append(accs[j - 1] + dv * dv)
                return tuple(new)

            accs = lax.fori_loop(
                0, _D, cstep,
                tuple(jnp.zeros((_L,), jnp.float32) for _ in range(_M - 1)))

            best = accs[0]
            bidx = jnp.full((_L,), 1.0, jnp.float32)
            for j in range(2, _M):
                upd = (nb >= j) & (accs[j - 1] < best)
                best = jnp.where(upd, accs[j - 1], best)
                bidx = jnp.where(upd, jnp.float32(j), bidx)
            nbf = nb.astype(jnp.float32)
            out_all[pl.ds(n * _C + g * _L, _L)] = \
                (bidx - 1.0) / (nbf - 1.0) + 1.0

    nbcp.wait()
    issue(0, 0)
    issue(1, 1)

    def step(k, carry):
        n0 = 2 * k
        drain(0)
        compute(n0, 0)
        issue(n0 + 2, 0)
        drain(1)
        compute(n0 + 1, 1)
        issue(n0 + 3, 1)
        return carry

    lax.fori_loop(0, (_NCHUNK - 2) // 2, step, 0)
    drain(0)
    compute(_NCHUNK - 2, 0)
    drain(1)
    compute(_NCHUNK - 1, 1)
    pltpu.sync_copy(out_all, out.at[pl.ds(base0, _PER_W)])


@jax.jit
def kernel(user_ids, item_ids, concept_ids, users_w, item_resp_w,
           nb_modalities, mask):
    del concept_ids, mask
    items_rows = _to_rows128(jnp.swapaxes(item_resp_w, 0, 1))
    users_rows = _to_rows128(jnp.swapaxes(users_w, 0, 1))
    run = pl.kernel(
        _impact_body,
        out_type=jax.ShapeDtypeStruct((_B,), jnp.float32),
        mesh=plsc.VectorSubcoreMesh(core_axis_name="c", subcore_axis_name="s",
                                    num_cores=_NC, num_subcores=_NS),
        compiler_params=pltpu.CompilerParams(needs_layout_passes=False,
                                             use_tc_tiling_on_sc=True),
        scratch_types=[
            pltpu.VMEM((_PER_W,), jnp.int32),
            pltpu.VMEM((_PER_W,), jnp.int32),
            pltpu.VMEM((_PER_W,), jnp.int32),
            pltpu.VMEM((_PER_W,), jnp.float32),
            pltpu.VMEM((2 * (_RPQ * _C + _C),), jnp.int32),
            pltpu.VMEM((_C, 128), jnp.float32),
            pltpu.VMEM((_C, 128), jnp.float32),
            pltpu.VMEM((_RPQ * _C, 128), jnp.float32),
            pltpu.VMEM((_RPQ * _C, 128), jnp.float32),
            pltpu.SemaphoreType.DMA,
            pltpu.SemaphoreType.DMA,
            pltpu.SemaphoreType.DMA,
        ],
    )
    return run(user_ids.astype(jnp.int32), item_ids.astype(jnp.int32),
               users_rows, items_rows, nb_modalities.astype(jnp.int32))


# speedup vs baseline: 1.1788x; 1.1788x over previous
import functools

import jax
import jax.numpy as jnp
from jax import lax
from jax.experimental import pallas as pl
from jax.experimental.pallas import tpu as pltpu
from jax.experimental.pallas import tpu_sc as plsc

_B = 16384
_M = 14
_D = 64
_NC = 2
_NS = 16
_L = 16
_NW = _NC * _NS
_PER_W = _B // _NW
_C = 32
_NCHUNK = _PER_W // _C
_NG = _C // _L
_RPQ = _M * _D // 128


def _tr_body(src, dst):
    x = src[...]
    xt = jnp.swapaxes(x, 0, 1)
    dst[...] = jnp.concatenate([xt[: _TS // 2], xt[_TS // 2:]], axis=1)


def _to_rows128(table_t):
    n = table_t.shape[1]
    grid = (n + _TS - 1) // _TS
    return pl.pallas_call(
        _tr_body,
        grid=(grid,),
        in_specs=[pl.BlockSpec((_D, _TS), lambda i: (0, i))],
        out_specs=pl.BlockSpec((_TS // 2, 128), lambda i: (i, 0)),
        out_shape=jax.ShapeDtypeStruct((grid * (_TS // 2), 128), jnp.float32),
    )(table_t)


def _impact_body(uids, iids, users, items, nbs, out,
                 uidx_all, iidx_all, nb_all, out_all, eidx,
                 u0, u1, e0, e1, sem_nb, sem0, sem1):
    wid = lax.axis_index("s") * _NC + lax.axis_index("c")
    base0 = wid * _PER_W
    iota = lax.iota(jnp.int32, _L)
    ubufs = (u0, u1)
    ebufs = (e0, e1)
    sems = (sem0, sem1)

    pltpu.sync_copy(uids.at[pl.ds(base0, _PER_W)], uidx_all)
    pltpu.sync_copy(iids.at[pl.ds(base0, _PER_W)], iidx_all)
    nbcp = pltpu.async_copy(nbs.at[iidx_all], nb_all, sem_nb)

    _SLOT = _RPQ * _C + _C

    def issue(n, s):
        for g in range(_NG):
            iv = iidx_all[pl.ds(n * _C + g * _L, _L)] * _RPQ
            for t in range(_RPQ):
                eidx[pl.ds(s * _SLOT + t * _C + g * _L, _L)] = iv + t
            uv = uidx_all[pl.ds(n * _C + g * _L, _L)]
            eidx[pl.ds(s * _SLOT + _RPQ * _C + g * _L, _L)] = uv >> 1
        pltpu.async_copy(items.at[eidx.at[pl.ds(s * _SLOT, _RPQ * _C)]],
                         ebufs[s], sems[s])
        pltpu.async_copy(users.at[eidx.at[pl.ds(s * _SLOT + _RPQ * _C, _C)]],
                         ubufs[s], sems[s])

    def drain(s):
        pltpu.make_async_copy(items.at[eidx.at[pl.ds(0, _RPQ * _C)]],
                              ebufs[s], sems[s]).wait()
        pltpu.make_async_copy(users.at[eidx.at[pl.ds(0, _C)]],
                              ubufs[s], sems[s]).wait()

    def compute(n, s):
        urows_v = ubufs[s]
        erows_v = ebufs[s]
        for g in range(_NG):
            rows = iota + (g * _L)
            nb = nb_all[pl.ds(n * _C + g * _L, _L)]
            uhalf = (uidx_all[pl.ds(n * _C + g * _L, _L)] & 1) << 6

            def cstep(c, accs, rows=rows, uhalf=uhalf,
                      urows_v=urows_v, erows_v=erows_v):
                csplat = jnp.full((_L,), 0, jnp.int32) + c
                u_c = plsc.load_gather(urows_v, [rows, uhalf + csplat])
                new = []
                for j in range(1, _M):
                    e = plsc.load_gather(
                        erows_v,
                        [rows + ((j // 2) * _C), csplat + ((j & 1) * _D)])
                    dv = u_c - e
                    new.append(accs[j - 1] + dv * dv)
                return tuple(new)

            accs = lax.fori_loop(
                0, _D, cstep,
                tuple(jnp.zeros((_L,), jnp.float32) for _ in range(_M - 1)))

            best = accs[0]
            bidx = jnp.full((_L,), 1.0, jnp.float32)
            for j in range(2, _M):
                upd = (nb >= j) & (accs[j - 1] < best)
                best = jnp.where(upd, accs[j - 1], best)
                bidx = jnp.where(upd, jnp.float32(j), bidx)
            nbf = nb.astype(jnp.float32)
            out_all[pl.ds(n * _C + g * _L, _L)] = \
                (bidx - 1.0) / (nbf - 1.0) + 1.0

    nbcp.wait()
    issue(0, 0)
    issue(1, 1)

    def step(k, carry):
        n0 = 2 * k
        drain(0)
        compute(n0, 0)
        issue(n0 + 2, 0)
        drain(1)
        compute(n0 + 1, 1)
        issue(n0 + 3, 1)
        return carry

    lax.fori_loop(0, (_NCHUNK - 2) // 2, step, 0)
    drain(0)
    compute(_NCHUNK - 2, 0)
    drain(1)
    compute(_NCHUNK - 1, 1)
    pltpu.sync_copy(out_all, out.at[pl.ds(base0, _PER_W)])


@jax.jit
def kernel(user_ids, item_ids, concept_ids, users_w, item_resp_w,
           nb_modalities, mask):
    del concept_ids, mask
    items_rows = item_resp_w.reshape(-1, 128)
    users_rows = users_w.reshape(-1, 128)
    run = pl.kernel(
        _impact_body,
        out_type=jax.ShapeDtypeStruct((_B,), jnp.float32),
        mesh=plsc.VectorSubcoreMesh(core_axis_name="c", subcore_axis_name="s",
                                    num_cores=_NC, num_subcores=_NS),
        compiler_params=pltpu.CompilerParams(needs_layout_passes=False,
                                             use_tc_tiling_on_sc=True),
        scratch_types=[
            pltpu.VMEM((_PER_W,), jnp.int32),
            pltpu.VMEM((_PER_W,), jnp.int32),
            pltpu.VMEM((_PER_W,), jnp.int32),
            pltpu.VMEM((_PER_W,), jnp.float32),
            pltpu.VMEM((2 * (_RPQ * _C + _C),), jnp.int32),
            pltpu.VMEM((_C, 128), jnp.float32),
            pltpu.VMEM((_C, 128), jnp.float32),
            pltpu.VMEM((_RPQ * _C, 128), jnp.float32),
            pltpu.VMEM((_RPQ * _C, 128), jnp.float32),
            pltpu.SemaphoreType.DMA,
            pltpu.SemaphoreType.DMA,
            pltpu.SemaphoreType.DMA,
        ],
    )
    return run(user_ids.astype(jnp.int32), item_ids.astype(jnp.int32),
               users_rows, items_rows, nb_modalities.astype(jnp.int32))
